# baseline (speedup 1.0000x reference)
import functools

import jax
import jax.numpy as jnp
from jax import lax
from jax.experimental import pallas as pl
from jax.experimental.pallas import tpu as pltpu
from jax.experimental.pallas import tpu_sc as plsc

NUM_DIVISION = 10
NUM_NODES = 100000
D_FEAT = 128
TAU = 1.0
CHUNK = NUM_NODES // NUM_DIVISION

TOTAL = NUM_NODES * D_FEAT
CHUNK_ELEMS = CHUNK * D_FEAT
HALF_STAGE = CHUNK_ELEMS // 4
NUM_STAGES = 20
SP_NBUF = 4
SLAB_PER_TILE = CHUNK_ELEMS // 32
VEC_ITERS = SLAB_PER_TILE // 16
IDX_PER_WORKER = 400
IDX_WORKERS = CHUNK // IDX_PER_WORKER

_MESH = plsc.VectorSubcoreMesh(core_axis_name="c", subcore_axis_name="s")


def _sc_body(idx_hbm, scale_hbm, x_hbm, out_hbm, outidx_hbm,
             idx_v, scale_v, oi_buf, slab_buf, sp0, sp1, sp2, sp3,
             slab_sem, sp_in_sems, sp_out_sems):
    c = lax.axis_index("c")
    s = lax.axis_index("s")
    w = s * 2 + c
    sps = (sp0, sp1, sp2, sp3)

    pltpu.sync_copy(idx_hbm, idx_v)
    pltpu.sync_copy(scale_hbm, scale_v)
    idx = idx_v[...][0]
    sv = scale_v[...]

    def stage_off(t):
        return (c * 5 + t // 4) * CHUNK_ELEMS + (t % 4) * HALF_STAGE

    conds = [(c * 5 + t // 4) != idx for t in range(NUM_STAGES)]

    def sp_in(t):
        b = t % SP_NBUF
        return pltpu.make_async_copy(
            x_hbm.at[pl.ds(stage_off(t), HALF_STAGE)], sps[b], sp_in_sems.at[b]
        )

    def sp_out(t):
        b = t % SP_NBUF
        return pltpu.make_async_copy(
            sps[b], out_hbm.at[pl.ds(stage_off(t), HALF_STAGE)], sp_out_sems.at[b]
        )

    @pl.when(s == 0)
    def _():
        for t in range(SP_NBUF - 1):
            @pl.when(conds[t])
            def _(t=t):
                sp_in(t).start()

    slab_off = idx * CHUNK_ELEMS + w * SLAB_PER_TILE
    pltpu.async_copy(
        x_hbm.at[pl.ds(slab_off, SLAB_PER_TILE)], slab_buf, slab_sem
    ).wait()

    def mbody(i, carry):
        sl = pl.ds(i * 16, 16)
        slab_buf[sl] = slab_buf[sl] * sv
        return carry

    lax.fori_loop(0, VEC_ITERS, mbody, 0)
    slab_out = pltpu.async_copy(
        slab_buf, out_hbm.at[pl.ds(slab_off, SLAB_PER_TILE)], slab_sem
    )

    @pl.when(w < IDX_WORKERS)
    def _():
        base = idx * CHUNK + w * IDX_PER_WORKER
        iota = lax.iota(jnp.int32, 16)

        def body(i, carry):
            oi_buf[pl.ds(i * 16, 16)] = base + i * 16 + iota
            return carry

        lax.fori_loop(0, IDX_PER_WORKER // 16, body, 0)
        pltpu.sync_copy(
            oi_buf, outidx_hbm.at[pl.ds(w * IDX_PER_WORKER, IDX_PER_WORKER)]
        )

    @pl.when(s == 0)
    def _():
        for t in range(NUM_STAGES):
            @pl.when(conds[t])
            def _(t=t):
                sp_in(t).wait()
                sp_out(t).start()

            nt = t + SP_NBUF - 1
            if nt < NUM_STAGES:
                pt = nt - SP_NBUF
                if pt >= 0:
                    @pl.when(conds[pt])
                    def _(pt=pt):
                        sp_out(pt).wait()

                @pl.when(conds[nt])
                def _(nt=nt):
                    sp_in(nt).start()

        for t in range(NUM_STAGES - SP_NBUF, NUM_STAGES):
            @pl.when(conds[t])
            def _(t=t):
                sp_out(t).wait()

    slab_out.wait()


_sc_copy_scale = functools.partial(
    pl.kernel,
    out_type=[
        jax.ShapeDtypeStruct((TOTAL,), jnp.float32),
        jax.ShapeDtypeStruct((CHUNK,), jnp.int32),
    ],
    mesh=_MESH,
    scratch_types=[
        pltpu.VMEM((16,), jnp.int32),
        pltpu.VMEM((16,), jnp.float32),
        pltpu.VMEM((IDX_PER_WORKER,), jnp.int32),
        pltpu.VMEM((SLAB_PER_TILE,), jnp.float32),
        pltpu.VMEM_SHARED((HALF_STAGE,), jnp.float32),
        pltpu.VMEM_SHARED((HALF_STAGE,), jnp.float32),
        pltpu.VMEM_SHARED((HALF_STAGE,), jnp.float32),
        pltpu.VMEM_SHARED((HALF_STAGE,), jnp.float32),
        pltpu.SemaphoreType.DMA,
        pltpu.SemaphoreType.DMA((SP_NBUF,)),
        pltpu.SemaphoreType.DMA((SP_NBUF,)),
    ],
)(_sc_body)


@jax.jit
def kernel(node_features, uniform_noise, sample_weights):
    g = -jnp.log(-jnp.log(uniform_noise))
    y_soft = jax.nn.softmax((sample_weights + g) / TAU, axis=-1)
    idx = jnp.argmax(y_soft, axis=-1).astype(jnp.int32)
    y = (1.0 + y_soft[idx]) - y_soft[idx]

    idx16 = jnp.full((16,), idx, dtype=jnp.int32)
    scale16 = jnp.full((16,), y, dtype=jnp.float32)
    x_flat = node_features.reshape(TOTAL)

    out_flat, outidx = _sc_copy_scale(idx16, scale16, x_flat)
    return out_flat.reshape(NUM_NODES, D_FEAT), outidx


# speedup vs baseline: 1.3699x; 1.3699x over previous
import functools

import jax
import jax.numpy as jnp
from jax import lax
from jax.experimental import pallas as pl
from jax.experimental.pallas import tpu as pltpu
from jax.experimental.pallas import tpu_sc as plsc

NUM_DIVISION = 10
NUM_NODES = 100000
D_FEAT = 128
TAU = 1.0
CHUNK = NUM_NODES // NUM_DIVISION

BLOCK_ROWS = 10000
NUM_BLOCKS = NUM_NODES // BLOCK_ROWS
BLOCKS_PER_CHUNK = max(1, CHUNK // BLOCK_ROWS)

IDX_PER_WORKER = 400
IDX_WORKERS = CHUNK // IDX_PER_WORKER

_MESH = plsc.VectorSubcoreMesh(core_axis_name="c", subcore_axis_name="s")


def _copy_scale_kernel(idx_ref, scale_ref, x_ref, out_ref):
    i = pl.program_id(0)
    in_slab = (i // BLOCKS_PER_CHUNK) == idx_ref[0]
    w = jnp.where(in_slab, scale_ref[0], jnp.float32(1.0))
    out_ref[...] = x_ref[...] * w


def _sc_indices_body(idx_hbm, outidx_hbm, idx_v, oi_buf):
    w = lax.axis_index("s") * 2 + lax.axis_index("c")

    @pl.when(w < IDX_WORKERS)
    def _():
        pltpu.sync_copy(idx_hbm, idx_v)
        idx = idx_v[...][0]
        base = idx * CHUNK + w * IDX_PER_WORKER
        iota = lax.iota(jnp.int32, 16)

        def body(i, carry):
            oi_buf[pl.ds(i * 16, 16)] = base + i * 16 + iota
            return carry

        lax.fori_loop(0, IDX_PER_WORKER // 16, body, 0)
        pltpu.sync_copy(
            oi_buf, outidx_hbm.at[pl.ds(w * IDX_PER_WORKER, IDX_PER_WORKER)]
        )


_sc_indices = functools.partial(
    pl.kernel,
    out_type=jax.ShapeDtypeStruct((CHUNK,), jnp.int32),
    mesh=_MESH,
    scratch_types=[
        pltpu.VMEM((16,), jnp.int32),
        pltpu.VMEM((IDX_PER_WORKER,), jnp.int32),
    ],
)(_sc_indices_body)


@jax.jit
def kernel(node_features, uniform_noise, sample_weights):
    g = -jnp.log(-jnp.log(uniform_noise))
    y_soft = jax.nn.softmax((sample_weights + g) / TAU, axis=-1)
    idx = jnp.argmax(y_soft, axis=-1).astype(jnp.int32)
    y = (1.0 + y_soft[idx]) - y_soft[idx]
    idx_arr = idx[None]
    scale_arr = y[None].astype(jnp.float32)
    idx16 = jnp.full((16,), idx, dtype=jnp.int32)

    updated = pl.pallas_call(
        _copy_scale_kernel,
        grid=(NUM_BLOCKS,),
        in_specs=[
            pl.BlockSpec(memory_space=pltpu.SMEM),
            pl.BlockSpec(memory_space=pltpu.SMEM),
            pl.BlockSpec((BLOCK_ROWS, D_FEAT), lambda i: (i, 0)),
        ],
        out_specs=pl.BlockSpec((BLOCK_ROWS, D_FEAT), lambda i: (i, 0)),
        out_shape=jax.ShapeDtypeStruct((NUM_NODES, D_FEAT), jnp.float32),
        compiler_params=pltpu.CompilerParams(
            dimension_semantics=("arbitrary",),
        ),
    )(idx_arr, scale_arr, node_features)

    outidx = _sc_indices(idx16)
    return updated, outidx
